# Initial kernel scaffold; baseline (speedup 1.0000x reference)
import jax
import jax.numpy as jnp
from jax.experimental import pallas as pl


def kernel(queries, memory_bank):
    raise NotImplementedError("write your pallas kernel here")


# speedup vs baseline: 1.8620x; 1.8620x over previous
import functools

import jax
import jax.numpy as jnp
from jax import lax
from jax.experimental import pallas as pl
from jax.experimental.pallas import tpu as pltpu
from jax.experimental.pallas import tpu_sc as plsc

D_MODEL = 512
MEM_SIZE = 100000
BATCH = 4096
TOP_K = 16

C_MEM = 2048
MEM_PAD = 100352
MC = MEM_PAD // C_MEM
TQ = 256
QT = BATCH // TQ

_NEG_INF = float("-inf")


def _norm_body(x_ref, o_ref):
    x = x_ref[...]
    n = jnp.sqrt(jnp.sum(x * x, axis=-1, keepdims=True))
    o_ref[...] = x / jnp.maximum(n, 1e-12)


def _normalize_rows(x, rows_per_blk, interpret=False):
    rows = x.shape[0]
    return pl.pallas_call(
        _norm_body,
        grid=(rows // rows_per_blk,),
        in_specs=[pl.BlockSpec((rows_per_blk, D_MODEL), lambda i: (i, 0))],
        out_specs=pl.BlockSpec((rows_per_blk, D_MODEL), lambda i: (i, 0)),
        out_shape=jax.ShapeDtypeStruct((rows, D_MODEL), jnp.float32),
        interpret=interpret,
    )(x)


def _topk_body(q_ref, m_ref, val_ref, idx_ref):
    mc = pl.program_id(1)
    sims = lax.dot_general(
        q_ref[...], m_ref[...],
        dimension_numbers=(((1,), (1,)), ((), ())),
        preferred_element_type=jnp.float32,
        precision=lax.Precision.DEFAULT,
    )
    col = mc * C_MEM + lax.broadcasted_iota(jnp.int32, (TQ, C_MEM), 1)
    bad = (sims > 0.999) | (sims < 0.0) | (col >= MEM_SIZE)
    sims = jnp.where(bad, _NEG_INF, sims)

    @pl.when(mc == 0)
    def _():
        val_ref[...] = jnp.full((TQ, TOP_K), _NEG_INF, jnp.float32)
        idx_ref[...] = jnp.full((TQ, TOP_K), jnp.int32(2**30), jnp.int32)

    comb_v = jnp.concatenate([sims, val_ref[...]], axis=1)
    comb_i = jnp.concatenate([col, idx_ref[...]], axis=1)
    vals, idxs = [], []
    big = jnp.int32(2**31 - 1)
    for _ in range(TOP_K):
        m = jnp.max(comb_v, axis=1, keepdims=True)
        sel = jnp.min(jnp.where(comb_v == m, comb_i, big), axis=1,
                      keepdims=True)
        vals.append(m)
        idxs.append(sel)
        comb_v = jnp.where(comb_i == sel, _NEG_INF, comb_v)
    val_ref[...] = jnp.concatenate(vals, axis=1)
    idx_ref[...] = jnp.concatenate(idxs, axis=1)


def _topk(q_n, mem_n, interpret=False):
    return pl.pallas_call(
        _topk_body,
        grid=(QT, MC),
        in_specs=[
            pl.BlockSpec((TQ, D_MODEL), lambda qt, mc: (qt, 0)),
            pl.BlockSpec((C_MEM, D_MODEL), lambda qt, mc: (mc, 0)),
        ],
        out_specs=[
            pl.BlockSpec((TQ, TOP_K), lambda qt, mc: (qt, 0)),
            pl.BlockSpec((TQ, TOP_K), lambda qt, mc: (qt, 0)),
        ],
        out_shape=[
            jax.ShapeDtypeStruct((BATCH, TOP_K), jnp.float32),
            jax.ShapeDtypeStruct((BATCH, TOP_K), jnp.int32),
        ],
        interpret=interpret,
    )(q_n, mem_n)


_NC, _NS = 2, 16
_NW = _NC * _NS
_GROWS = BATCH * TOP_K
_BPW = _GROWS // _NW
_CH = 128
_NCH = _BPW // _CH


def _sc_gather_body(table_hbm, idx_hbm, out_hbm, idx_v, rows_v, sem):
    wid = lax.axis_index("s") * _NC + lax.axis_index("c")
    base = wid * _BPW

    def step(j, carry):
        off = pl.multiple_of(base + j * _CH, _CH)
        pltpu.sync_copy(idx_hbm.at[pl.ds(off, _CH)], idx_v)
        pltpu.async_copy(table_hbm.at[idx_v], rows_v, sem).wait()
        pltpu.sync_copy(rows_v, out_hbm.at[pl.ds(off, _CH)])
        return carry

    lax.fori_loop(0, _NCH, step, 0)


def _sc_gather(mem_n, flat_idx):
    mesh = plsc.VectorSubcoreMesh(core_axis_name="c", subcore_axis_name="s")
    run = functools.partial(
        pl.kernel,
        mesh=mesh,
        out_type=jax.ShapeDtypeStruct((_GROWS, D_MODEL), jnp.float32),
        scratch_types=[
            pltpu.VMEM((_CH,), jnp.int32),
            pltpu.VMEM((_CH, D_MODEL), jnp.float32),
            pltpu.SemaphoreType.DMA,
        ],
    )(_sc_gather_body)
    return run(mem_n, flat_idx)


def kernel(queries, memory_bank):
    memp = jnp.pad(memory_bank, ((0, MEM_PAD - MEM_SIZE), (0, 0)))
    mem_n = _normalize_rows(memp, 2048)
    q_n = _normalize_rows(queries, 512)
    top_sims, top_idx = _topk(q_n, mem_n)
    retrieved = _sc_gather(mem_n, top_idx.reshape(-1))
    retrieved = retrieved.reshape(BATCH, TOP_K, D_MODEL)
    mask = top_sims > _NEG_INF
    return retrieved, top_sims, mask


# speedup vs baseline: 4.4798x; 2.4059x over previous
import functools

import jax
import jax.numpy as jnp
from jax import lax
from jax.experimental import pallas as pl
from jax.experimental.pallas import tpu as pltpu
from jax.experimental.pallas import tpu_sc as plsc

D_MODEL = 512
MEM_SIZE = 100000
BATCH = 4096
TOP_K = 16

C_MEM = 2048
MEM_PAD = 100352
MC = MEM_PAD // C_MEM
TQ = 256
QT = BATCH // TQ
G = 128
NG = MEM_PAD // G
NG_C = C_MEM // G
CAND = TOP_K * G

_NEG_INF = float("-inf")


def _norm_body(x_ref, o_ref):
    x = x_ref[...]
    n = jnp.sqrt(jnp.sum(x * x, axis=-1, keepdims=True))
    o_ref[...] = x / jnp.maximum(n, 1e-12)


def _normalize_rows(x, rows_per_blk, interpret=False):
    rows = x.shape[0]
    return pl.pallas_call(
        _norm_body,
        grid=(rows // rows_per_blk,),
        in_specs=[pl.BlockSpec((rows_per_blk, D_MODEL), lambda i: (i, 0))],
        out_specs=pl.BlockSpec((rows_per_blk, D_MODEL), lambda i: (i, 0)),
        out_shape=jax.ShapeDtypeStruct((rows, D_MODEL), jnp.float32),
        interpret=interpret,
    )(x)


def _matmul_gmax_body(q_ref, m_ref, sims_ref, gmax_ref):
    mc = pl.program_id(1)
    sims = lax.dot_general(
        q_ref[...], m_ref[...],
        dimension_numbers=(((1,), (1,)), ((), ())),
        preferred_element_type=jnp.float32,
        precision=lax.Precision.DEFAULT,
    )
    col = mc * C_MEM + lax.broadcasted_iota(jnp.int32, (TQ, C_MEM), 1)
    bad = (sims > 0.999) | (sims < 0.0) | (col >= MEM_SIZE)
    sims = jnp.where(bad, _NEG_INF, sims)
    sims_ref[...] = sims
    gmax_ref[...] = jnp.max(sims.reshape(TQ, NG_C, G), axis=-1)[None]


def _matmul_gmax(q_n, mem_n, interpret=False):
    return pl.pallas_call(
        _matmul_gmax_body,
        grid=(QT, MC),
        in_specs=[
            pl.BlockSpec((TQ, D_MODEL), lambda qt, mc: (qt, 0)),
            pl.BlockSpec((C_MEM, D_MODEL), lambda qt, mc: (mc, 0)),
        ],
        out_specs=[
            pl.BlockSpec((TQ, C_MEM), lambda qt, mc: (qt, mc)),
            pl.BlockSpec((1, TQ, NG_C), lambda qt, mc: (mc, qt, 0)),
        ],
        out_shape=[
            jax.ShapeDtypeStruct((BATCH, MEM_PAD), jnp.float32),
            jax.ShapeDtypeStruct((MC, BATCH, NG_C), jnp.float32),
        ],
        interpret=interpret,
    )(q_n, mem_n)


def _extract_topk(vals, ids):
    big = jnp.int32(2**31 - 1)
    out_v, out_i = [], []
    for _ in range(TOP_K):
        m = jnp.max(vals, axis=1, keepdims=True)
        sel = jnp.min(jnp.where(vals == m, ids, big), axis=1, keepdims=True)
        out_v.append(m)
        out_i.append(sel)
        vals = jnp.where(ids == sel, _NEG_INF, vals)
    return jnp.concatenate(out_v, axis=1), jnp.concatenate(out_i, axis=1)


TQ3 = 1024


def _group_select_body(gmax_ref, grow_ref):
    qt = pl.program_id(0)
    gid = lax.broadcasted_iota(jnp.int32, (TQ3, NG), 1)
    _, sel = _extract_topk(gmax_ref[...], gid)
    row = qt * TQ3 + lax.broadcasted_iota(jnp.int32, (TQ3, TOP_K), 0)
    grow_ref[...] = row * NG + sel


def _group_select(gmax, interpret=False):
    return pl.pallas_call(
        _group_select_body,
        grid=(BATCH // TQ3,),
        in_specs=[pl.BlockSpec((TQ3, NG), lambda i: (i, 0))],
        out_specs=pl.BlockSpec((TQ3, TOP_K), lambda i: (i, 0)),
        out_shape=jax.ShapeDtypeStruct((BATCH, TOP_K), jnp.int32),
        interpret=interpret,
    )(gmax)


TQ5 = 512


def _final_select_body(cand_ref, grow_ref, val_ref, idx_ref):
    qt = pl.program_id(0)
    row = qt * TQ5 + lax.broadcasted_iota(jnp.int32, (TQ5, 1), 0)
    lane = lax.broadcasted_iota(jnp.int32, (TQ5, G), 1)
    cols = []
    for k in range(TOP_K):
        g = grow_ref[...][:, k:k + 1] - row * NG
        cols.append(g * G + lane)
    colfull = jnp.concatenate(cols, axis=1)
    v, i = _extract_topk(cand_ref[...], colfull)
    val_ref[...] = v
    idx_ref[...] = i


def _final_select(cand, grow, interpret=False):
    return pl.pallas_call(
        _final_select_body,
        grid=(BATCH // TQ5,),
        in_specs=[
            pl.BlockSpec((TQ5, CAND), lambda i: (i, 0)),
            pl.BlockSpec((TQ5, TOP_K), lambda i: (i, 0)),
        ],
        out_specs=[
            pl.BlockSpec((TQ5, TOP_K), lambda i: (i, 0)),
            pl.BlockSpec((TQ5, TOP_K), lambda i: (i, 0)),
        ],
        out_shape=[
            jax.ShapeDtypeStruct((BATCH, TOP_K), jnp.float32),
            jax.ShapeDtypeStruct((BATCH, TOP_K), jnp.int32),
        ],
        interpret=interpret,
    )(cand, grow)


_NC, _NS = 2, 16
_NW = _NC * _NS
_GROWS = BATCH * TOP_K
_BPW = _GROWS // _NW


def _sc_gather_body(ch, nch, table_hbm, idx_hbm, out_hbm, idx_v, rows_v, sem):
    wid = lax.axis_index("s") * _NC + lax.axis_index("c")
    base = wid * _BPW

    def step(j, carry):
        off = pl.multiple_of(base + j * ch, ch)
        pltpu.sync_copy(idx_hbm.at[pl.ds(off, ch)], idx_v)
        pltpu.async_copy(table_hbm.at[idx_v], rows_v, sem).wait()
        pltpu.sync_copy(rows_v, out_hbm.at[pl.ds(off, ch)])
        return carry

    lax.fori_loop(0, nch, step, 0)


def _sc_gather(table, flat_idx, ch):
    d = table.shape[1]
    mesh = plsc.VectorSubcoreMesh(core_axis_name="c", subcore_axis_name="s")
    run = functools.partial(
        pl.kernel,
        mesh=mesh,
        out_type=jax.ShapeDtypeStruct((_GROWS, d), jnp.float32),
        scratch_types=[
            pltpu.VMEM((ch,), jnp.int32),
            pltpu.VMEM((ch, d), jnp.float32),
            pltpu.SemaphoreType.DMA,
        ],
    )(functools.partial(_sc_gather_body, ch, _BPW // ch))
    return run(table, flat_idx)


def kernel(queries, memory_bank):
    memp = jnp.pad(memory_bank, ((0, MEM_PAD - MEM_SIZE), (0, 0)))
    mem_n = _normalize_rows(memp, 2048)
    q_n = _normalize_rows(queries, 512)
    sims, gmax3 = _matmul_gmax(q_n, mem_n)
    gmax = gmax3.transpose(1, 0, 2).reshape(BATCH, NG)
    grow = _group_select(gmax)
    cand = _sc_gather(sims.reshape(BATCH * NG, G), grow.reshape(-1), ch=512)
    top_sims, top_idx = _final_select(cand.reshape(BATCH, CAND), grow)
    retrieved = _sc_gather(mem_n, top_idx.reshape(-1), ch=128)
    retrieved = retrieved.reshape(BATCH, TOP_K, D_MODEL)
    mask = top_sims > _NEG_INF
    return retrieved, top_sims, mask


# speedup vs baseline: 4.8951x; 1.0927x over previous
import functools

import jax
import jax.numpy as jnp
from jax import lax
from jax.experimental import pallas as pl
from jax.experimental.pallas import tpu as pltpu
from jax.experimental.pallas import tpu_sc as plsc

D_MODEL = 512
MEM_SIZE = 100000
BATCH = 4096
TOP_K = 16

C_MEM = 2048
MEM_PAD = 100352
MC = MEM_PAD // C_MEM
TQ = 256
QT = BATCH // TQ
G = 128
NG = MEM_PAD // G
NG_C = C_MEM // G
CAND = TOP_K * G

_NEG_INF = float("-inf")


def _normalize_rows(x):
    n = jnp.sqrt(jnp.sum(x * x, axis=-1, keepdims=True))
    return x / jnp.maximum(n, 1e-12)


def _matmul_gmax_body(q_ref, m_ref, sims_ref, gmax_ref):
    mc = pl.program_id(0)
    sims = lax.dot_general(
        q_ref[...], m_ref[...],
        dimension_numbers=(((1,), (1,)), ((), ())),
        preferred_element_type=jnp.float32,
        precision=lax.Precision.DEFAULT,
    )
    col = mc * C_MEM + lax.broadcasted_iota(jnp.int32, (TQ, C_MEM), 1)
    bad = (sims > 0.999) | (sims < 0.0) | (col >= MEM_SIZE)
    sims = jnp.where(bad, _NEG_INF, sims)
    sims_ref[...] = sims
    gmax_ref[...] = jnp.max(sims.reshape(TQ, NG_C, G), axis=-1)[None]


def _matmul_gmax(q_n, mem_n, interpret=False):
    return pl.pallas_call(
        _matmul_gmax_body,
        grid=(MC, QT),
        in_specs=[
            pl.BlockSpec((TQ, D_MODEL), lambda mc, qt: (qt, 0)),
            pl.BlockSpec((C_MEM, D_MODEL), lambda mc, qt: (mc, 0)),
        ],
        out_specs=[
            pl.BlockSpec((TQ, C_MEM), lambda mc, qt: (qt, mc)),
            pl.BlockSpec((1, TQ, NG_C), lambda mc, qt: (mc, qt, 0)),
        ],
        out_shape=[
            jax.ShapeDtypeStruct((BATCH, MEM_PAD), jnp.float32),
            jax.ShapeDtypeStruct((MC, BATCH, NG_C), jnp.float32),
        ],
        interpret=interpret,
    )(q_n, mem_n)


def _extract_topk(vals, ids):
    big = jnp.int32(2**31 - 1)
    out_v, out_i = [], []
    for _ in range(TOP_K):
        m = jnp.max(vals, axis=1, keepdims=True)
        sel = jnp.min(jnp.where(vals == m, ids, big), axis=1, keepdims=True)
        out_v.append(m)
        out_i.append(sel)
        vals = jnp.where(ids == sel, _NEG_INF, vals)
    return jnp.concatenate(out_v, axis=1), jnp.concatenate(out_i, axis=1)


TQ3 = 1024


def _group_select_body(gmax_ref, grow_ref):
    qt = pl.program_id(0)
    gid = lax.broadcasted_iota(jnp.int32, (TQ3, NG), 1)
    _, sel = _extract_topk(gmax_ref[...], gid)
    row = qt * TQ3 + lax.broadcasted_iota(jnp.int32, (TQ3, TOP_K), 0)
    grow_ref[...] = row * NG + sel


def _group_select(gmax, interpret=False):
    return pl.pallas_call(
        _group_select_body,
        grid=(BATCH // TQ3,),
        in_specs=[pl.BlockSpec((TQ3, NG), lambda i: (i, 0))],
        out_specs=pl.BlockSpec((TQ3, TOP_K), lambda i: (i, 0)),
        out_shape=jax.ShapeDtypeStruct((BATCH, TOP_K), jnp.int32),
        interpret=interpret,
    )(gmax)


TQ5 = 512


def _final_select_body(cand_ref, grow_ref, val_ref, idx_ref):
    qt = pl.program_id(0)
    row = qt * TQ5 + lax.broadcasted_iota(jnp.int32, (TQ5, 1), 0)
    lane = lax.broadcasted_iota(jnp.int32, (TQ5, G), 1)
    cols = []
    for k in range(TOP_K):
        g = grow_ref[...][:, k:k + 1] - row * NG
        cols.append(g * G + lane)
    colfull = jnp.concatenate(cols, axis=1)
    v, i = _extract_topk(cand_ref[...], colfull)
    val_ref[...] = v
    idx_ref[...] = i


def _final_select(cand, grow, interpret=False):
    return pl.pallas_call(
        _final_select_body,
        grid=(BATCH // TQ5,),
        in_specs=[
            pl.BlockSpec((TQ5, CAND), lambda i: (i, 0)),
            pl.BlockSpec((TQ5, TOP_K), lambda i: (i, 0)),
        ],
        out_specs=[
            pl.BlockSpec((TQ5, TOP_K), lambda i: (i, 0)),
            pl.BlockSpec((TQ5, TOP_K), lambda i: (i, 0)),
        ],
        out_shape=[
            jax.ShapeDtypeStruct((BATCH, TOP_K), jnp.float32),
            jax.ShapeDtypeStruct((BATCH, TOP_K), jnp.int32),
        ],
        interpret=interpret,
    )(cand, grow)


_NC, _NS = 2, 16
_NW = _NC * _NS
_GROWS = BATCH * TOP_K
_BPW = _GROWS // _NW


def _sc_gather_body(ch, nch, table_hbm, idx_hbm, out_hbm, idx_v, rows_v, sem):
    wid = lax.axis_index("s") * _NC + lax.axis_index("c")
    base = wid * _BPW

    def step(j, carry):
        off = pl.multiple_of(base + j * ch, ch)
        pltpu.sync_copy(idx_hbm.at[pl.ds(off, ch)], idx_v)
        pltpu.async_copy(table_hbm.at[idx_v], rows_v, sem).wait()
        pltpu.sync_copy(rows_v, out_hbm.at[pl.ds(off, ch)])
        return carry

    lax.fori_loop(0, nch, step, 0)


def _sc_gather(table, flat_idx, ch):
    d = table.shape[1]
    mesh = plsc.VectorSubcoreMesh(core_axis_name="c", subcore_axis_name="s")
    run = functools.partial(
        pl.kernel,
        mesh=mesh,
        out_type=jax.ShapeDtypeStruct((_GROWS, d), jnp.float32),
        scratch_types=[
            pltpu.VMEM((ch,), jnp.int32),
            pltpu.VMEM((ch, d), jnp.float32),
            pltpu.SemaphoreType.DMA,
        ],
    )(functools.partial(_sc_gather_body, ch, _BPW // ch))
    return run(table, flat_idx)


def kernel(queries, memory_bank):
    mem_n = jnp.pad(_normalize_rows(memory_bank), ((0, MEM_PAD - MEM_SIZE), (0, 0)))
    q_n = _normalize_rows(queries)
    sims, gmax3 = _matmul_gmax(q_n, mem_n)
    gmax = gmax3.transpose(1, 0, 2).reshape(BATCH, NG)
    grow = _group_select(gmax)
    cand = _sc_gather(sims.reshape(BATCH * NG, G), grow.reshape(-1), ch=512)
    top_sims, top_idx = _final_select(cand.reshape(BATCH, CAND), grow)
    retrieved = _sc_gather(mem_n, top_idx.reshape(-1), ch=128)
    retrieved = retrieved.reshape(BATCH, TOP_K, D_MODEL)
    mask = top_sims > _NEG_INF
    return retrieved, top_sims, mask
